# baseline (speedup 1.0000x reference)
import functools

import jax
import jax.numpy as jnp
from jax import lax
from jax.experimental import pallas as pl
from jax.experimental.pallas import tpu as pltpu
from jax.experimental.pallas import tpu_sc as plsc

_NC = 2
_NS = 16
_NW = _NC * _NS
_CH = 512
_PER_TILE = 3072
_LANE = 128


def _tc_body(i_ref, o_ref):
    o_ref[...] = i_ref[...]


def _tc_copy(x, blk):
    n = x.shape[1]
    grid = (n + blk - 1) // blk
    return pl.pallas_call(
        _tc_body,
        grid=(grid,),
        in_specs=[pl.BlockSpec((x.shape[0], blk), lambda g: (0, g))],
        out_specs=pl.BlockSpec((x.shape[0], blk), lambda g: (0, g)),
        out_shape=jax.ShapeDtypeStruct(x.shape, x.dtype),
    )(x)


def _tail_body(_, t_ref, o_ref):
    o_ref[...] = t_ref[...]


def _tail_patch(partial_copy, src, tail_block):
    d, n = src.shape
    return pl.pallas_call(
        _tail_body,
        grid=(1,),
        in_specs=[
            pl.BlockSpec(memory_space=pl.ANY),
            pl.BlockSpec((d, _LANE), lambda g: (0, tail_block)),
        ],
        out_specs=pl.BlockSpec((d, _LANE), lambda g: (0, tail_block)),
        out_shape=jax.ShapeDtypeStruct((d, n), src.dtype),
        input_output_aliases={0: 0},
    )(partial_copy, src)


def _sc_stream(src, dst, cols, bufs, isems, osems):
    def mk_in(idx, b):
        c0, w = cols[idx]
        return pltpu.make_async_copy(
            src.at[:, pl.ds(c0, w)], bufs[b].at[:, pl.ds(0, w)], isems.at[b])

    def mk_out(idx, b):
        c0, w = cols[idx]
        return pltpu.make_async_copy(
            bufs[b].at[:, pl.ds(0, w)], dst.at[:, pl.ds(c0, w)], osems.at[b])

    n = len(cols)
    mk_in(0, 0).start()
    for c in range(n):
        b = c & 1
        nb = 1 - b
        if c + 1 < n:
            if c >= 1:
                mk_out(c - 1, nb).wait()
            mk_in(c + 1, nb).start()
        mk_in(c, b).wait()
        mk_out(c, b).start()
    if n >= 2:
        mk_out(n - 2, (n - 2) & 1).wait()
    mk_out(n - 1, (n - 1) & 1).wait()


def _sc_body(i_in, i_out, buf0, buf1, isems, osems):
    wid = lax.axis_index("s") * _NC + lax.axis_index("c")
    n = i_in.shape[1]
    aligned = (n // _LANE) * _LANE
    extra0 = _NW * _PER_TILE
    n_extra = (aligned - extra0) // _LANE

    base = wid * _PER_TILE
    cols = [(base + k * _CH, _CH) for k in range(_PER_TILE // _CH)]
    bufs = (buf0, buf1)
    _sc_stream(i_in, i_out, cols, bufs, isems, osems)

    @pl.when(wid < n_extra)
    def _extras():
        c0 = extra0 + wid * _LANE
        _sc_stream(i_in, i_out, [(c0, _LANE)], bufs, isems, osems)


def _sc_copy(x):
    mesh = plsc.VectorSubcoreMesh(core_axis_name="c", subcore_axis_name="s")
    run = functools.partial(
        pl.kernel,
        mesh=mesh,
        out_type=jax.ShapeDtypeStruct(x.shape, x.dtype),
        scratch_types=[
            pltpu.VMEM((x.shape[0], _CH), jnp.float32),
            pltpu.VMEM((x.shape[0], _CH), jnp.float32),
            pltpu.SemaphoreType.DMA((2,)),
            pltpu.SemaphoreType.DMA((2,)),
        ],
    )(_sc_body)
    return run(x)


def kernel(perturb, user_emb, item_emb):
    del perturb
    it = item_emb.T
    i_bulk = _sc_copy(it)
    u = _tc_copy(user_emb.T, 16000).T
    i = _tail_patch(i_bulk, it, it.shape[1] // _LANE).T
    return (u, i)


# speedup vs baseline: 1.0137x; 1.0137x over previous
import functools

import jax
import jax.numpy as jnp
from jax import lax
from jax.experimental import pallas as pl
from jax.experimental.pallas import tpu as pltpu
from jax.experimental.pallas import tpu_sc as plsc

_NC = 2
_NS = 16
_NW = _NC * _NS
_CH = 512
_PER_TILE = 3072
_LANE = 128


def _tc_body(i_ref, o_ref):
    o_ref[...] = i_ref[...]


def _tc_copy(x, blk):
    n = x.shape[1]
    grid = (n + blk - 1) // blk
    return pl.pallas_call(
        _tc_body,
        grid=(grid,),
        in_specs=[pl.BlockSpec((x.shape[0], blk), lambda g: (0, g))],
        out_specs=pl.BlockSpec((x.shape[0], blk), lambda g: (0, g)),
        out_shape=jax.ShapeDtypeStruct(x.shape, x.dtype),
    )(x)


def _tail_body(_, t_ref, o_ref):
    o_ref[...] = t_ref[...]


def _tail_patch(partial_copy, src, tail_block):
    d, n = src.shape
    return pl.pallas_call(
        _tail_body,
        grid=(1,),
        in_specs=[
            pl.BlockSpec(memory_space=pl.ANY),
            pl.BlockSpec((d, _LANE), lambda g: (0, tail_block)),
        ],
        out_specs=pl.BlockSpec((d, _LANE), lambda g: (0, tail_block)),
        out_shape=jax.ShapeDtypeStruct((d, n), src.dtype),
        input_output_aliases={0: 0},
    )(partial_copy, src)


def _sc_stream(src, dst, cols, bufs, isems, osems):
    def mk_in(idx, b):
        c0, w = cols[idx]
        return pltpu.make_async_copy(
            src.at[:, pl.ds(c0, w)], bufs[b].at[:, pl.ds(0, w)], isems.at[b])

    def mk_out(idx, b):
        c0, w = cols[idx]
        return pltpu.make_async_copy(
            bufs[b].at[:, pl.ds(0, w)], dst.at[:, pl.ds(c0, w)], osems.at[b])

    n = len(cols)
    mk_in(0, 0).start()
    for c in range(n):
        b = c & 1
        nb = 1 - b
        if c + 1 < n:
            if c >= 1:
                mk_out(c - 1, nb).wait()
            mk_in(c + 1, nb).start()
        mk_in(c, b).wait()
        mk_out(c, b).start()
    if n >= 2:
        mk_out(n - 2, (n - 2) & 1).wait()
    mk_out(n - 1, (n - 1) & 1).wait()


def _sc_body(i_in, i_out, buf0, buf1, isems, osems):
    wid = lax.axis_index("s") * _NC + lax.axis_index("c")
    n = i_in.shape[1]
    aligned = (n // _LANE) * _LANE
    extra0 = _NW * _PER_TILE
    n_extra = (aligned - extra0) // _LANE

    base = wid * _PER_TILE
    cols = [(base + k * _CH, _CH) for k in range(_PER_TILE // _CH)]
    bufs = (buf0, buf1)
    _sc_stream(i_in, i_out, cols, bufs, isems, osems)

    @pl.when(wid < n_extra)
    def _extras():
        c0 = extra0 + wid * _LANE
        _sc_stream(i_in, i_out, [(c0, _LANE)], bufs, isems, osems)


def _sc_copy(x):
    mesh = plsc.VectorSubcoreMesh(core_axis_name="c", subcore_axis_name="s")
    run = functools.partial(
        pl.kernel,
        mesh=mesh,
        out_type=jax.ShapeDtypeStruct(x.shape, x.dtype),
        scratch_types=[
            pltpu.VMEM((x.shape[0], _CH), jnp.float32),
            pltpu.VMEM((x.shape[0], _CH), jnp.float32),
            pltpu.SemaphoreType.DMA((2,)),
            pltpu.SemaphoreType.DMA((2,)),
        ],
    )(_sc_body)
    return run(x)


def kernel(perturb, user_emb, item_emb):
    del perturb
    it = item_emb.T
    i_bulk = _sc_copy(it)
    u = _tc_copy(user_emb.T, 25088).T
    i = _tail_patch(i_bulk, it, it.shape[1] // _LANE).T
    return (u, i)


# speedup vs baseline: 1.0213x; 1.0075x over previous
import functools

import jax
import jax.numpy as jnp
from jax import lax
from jax.experimental import pallas as pl
from jax.experimental.pallas import tpu as pltpu
from jax.experimental.pallas import tpu_sc as plsc

_NC = 2
_NS = 16
_NW = _NC * _NS
_CH = 512
_PER_TILE = 3072
_LANE = 128


def _tc_body(i_ref, o_ref):
    o_ref[...] = i_ref[...]


def _tc_copy(x, blk):
    n = x.shape[1]
    grid = (n + blk - 1) // blk
    return pl.pallas_call(
        _tc_body,
        grid=(grid,),
        in_specs=[pl.BlockSpec((x.shape[0], blk), lambda g: (0, g))],
        out_specs=pl.BlockSpec((x.shape[0], blk), lambda g: (0, g)),
        out_shape=jax.ShapeDtypeStruct(x.shape, x.dtype),
    )(x)


def _tail_body(_, t_ref, o_ref):
    o_ref[...] = t_ref[...]


def _tail_patch(partial_copy, src, tail_block):
    d, n = src.shape
    return pl.pallas_call(
        _tail_body,
        grid=(1,),
        in_specs=[
            pl.BlockSpec(memory_space=pl.ANY),
            pl.BlockSpec((d, _LANE), lambda g: (0, tail_block)),
        ],
        out_specs=pl.BlockSpec((d, _LANE), lambda g: (0, tail_block)),
        out_shape=jax.ShapeDtypeStruct((d, n), src.dtype),
        input_output_aliases={0: 0},
    )(partial_copy, src)


def _sc_stream(src, dst, cols, bufs, isems, osems):
    def mk_in(idx, b):
        c0, w = cols[idx]
        return pltpu.make_async_copy(
            src.at[:, pl.ds(c0, w)], bufs[b].at[:, pl.ds(0, w)], isems.at[b])

    def mk_out(idx, b):
        c0, w = cols[idx]
        return pltpu.make_async_copy(
            bufs[b].at[:, pl.ds(0, w)], dst.at[:, pl.ds(c0, w)], osems.at[b])

    n = len(cols)
    mk_in(0, 0).start()
    for c in range(n):
        b = c & 1
        nb = 1 - b
        if c + 1 < n:
            if c >= 1:
                mk_out(c - 1, nb).wait()
            mk_in(c + 1, nb).start()
        mk_in(c, b).wait()
        mk_out(c, b).start()
    if n >= 2:
        mk_out(n - 2, (n - 2) & 1).wait()
    mk_out(n - 1, (n - 1) & 1).wait()


def _sc_body(i_in, i_out, buf0, buf1, isems, osems):
    wid = lax.axis_index("s") * _NC + lax.axis_index("c")
    n = i_in.shape[1]
    aligned = (n // _LANE) * _LANE
    extra0 = _NW * _PER_TILE
    n_extra = (aligned - extra0) // _LANE

    base = wid * _PER_TILE
    cols = [(base + k * _CH, _CH) for k in range(_PER_TILE // _CH)]
    bufs = (buf0, buf1)
    _sc_stream(i_in, i_out, cols, bufs, isems, osems)

    @pl.when(wid < n_extra)
    def _extras():
        c0 = extra0 + wid * _LANE
        _sc_stream(i_in, i_out, [(c0, _LANE)], bufs, isems, osems)


def _sc_copy(x):
    mesh = plsc.VectorSubcoreMesh(core_axis_name="c", subcore_axis_name="s")
    run = functools.partial(
        pl.kernel,
        mesh=mesh,
        out_type=jax.ShapeDtypeStruct(x.shape, x.dtype),
        scratch_types=[
            pltpu.VMEM((x.shape[0], _CH), jnp.float32),
            pltpu.VMEM((x.shape[0], _CH), jnp.float32),
            pltpu.SemaphoreType.DMA((2,)),
            pltpu.SemaphoreType.DMA((2,)),
        ],
    )(_sc_body)
    return run(x)


def kernel(perturb, user_emb, item_emb):
    del perturb
    it = item_emb.T
    i_bulk = _sc_copy(it)
    u = _tc_copy(user_emb.T, 50176).T
    i = _tail_patch(i_bulk, it, it.shape[1] // _LANE).T
    return (u, i)


# speedup vs baseline: 1.0222x; 1.0008x over previous
import functools

import jax
import jax.numpy as jnp
from jax import lax
from jax.experimental import pallas as pl
from jax.experimental.pallas import tpu as pltpu
from jax.experimental.pallas import tpu_sc as plsc

_NC = 2
_NS = 16
_NW = _NC * _NS
_CH = 512
_PER_TILE = 3072
_LANE = 128


def _tc_body(i_ref, o_ref):
    o_ref[...] = i_ref[...]


def _tc_copy(x, blk):
    n = x.shape[1]
    grid = (n + blk - 1) // blk
    return pl.pallas_call(
        _tc_body,
        grid=(grid,),
        in_specs=[pl.BlockSpec((x.shape[0], blk), lambda g: (0, g))],
        out_specs=pl.BlockSpec((x.shape[0], blk), lambda g: (0, g)),
        out_shape=jax.ShapeDtypeStruct(x.shape, x.dtype),
    )(x)


def _tail_body(_, t_ref, o_ref):
    o_ref[...] = t_ref[...]


def _tail_patch(partial_copy, src, tail_block):
    d, n = src.shape
    return pl.pallas_call(
        _tail_body,
        grid=(1,),
        in_specs=[
            pl.BlockSpec(memory_space=pl.ANY),
            pl.BlockSpec((d, _LANE), lambda g: (0, tail_block)),
        ],
        out_specs=pl.BlockSpec((d, _LANE), lambda g: (0, tail_block)),
        out_shape=jax.ShapeDtypeStruct((d, n), src.dtype),
        input_output_aliases={0: 0},
    )(partial_copy, src)


def _sc_stream(src, dst, cols, bufs, isems, osems):
    def mk_in(idx, b):
        c0, w = cols[idx]
        return pltpu.make_async_copy(
            src.at[:, pl.ds(c0, w)], bufs[b].at[:, pl.ds(0, w)], isems.at[b])

    def mk_out(idx, b):
        c0, w = cols[idx]
        return pltpu.make_async_copy(
            bufs[b].at[:, pl.ds(0, w)], dst.at[:, pl.ds(c0, w)], osems.at[b])

    n = len(cols)
    mk_in(0, 0).start()
    for c in range(n):
        b = c & 1
        nb = 1 - b
        if c + 1 < n:
            if c >= 1:
                mk_out(c - 1, nb).wait()
            mk_in(c + 1, nb).start()
        mk_in(c, b).wait()
        mk_out(c, b).start()
    if n >= 2:
        mk_out(n - 2, (n - 2) & 1).wait()
    mk_out(n - 1, (n - 1) & 1).wait()


def _sc_body(i_in, i_out, buf0, buf1, isems, osems):
    wid = lax.axis_index("s") * _NC + lax.axis_index("c")
    n = i_in.shape[1]
    aligned = (n // _LANE) * _LANE
    extra0 = _NW * _PER_TILE
    n_extra = (aligned - extra0) // _LANE

    base = wid * _PER_TILE
    cols = [(base + k * _CH, _CH) for k in range(_PER_TILE // _CH)]
    bufs = (buf0, buf1)
    _sc_stream(i_in, i_out, cols, bufs, isems, osems)

    @pl.when(wid < n_extra)
    def _extras():
        c0 = extra0 + wid * _LANE
        _sc_stream(i_in, i_out, [(c0, _LANE)], bufs, isems, osems)


def _sc_copy(x):
    mesh = plsc.VectorSubcoreMesh(core_axis_name="c", subcore_axis_name="s")
    run = functools.partial(
        pl.kernel,
        mesh=mesh,
        out_type=jax.ShapeDtypeStruct(x.shape, x.dtype),
        scratch_types=[
            pltpu.VMEM((x.shape[0], _CH), jnp.float32),
            pltpu.VMEM((x.shape[0], _CH), jnp.float32),
            pltpu.SemaphoreType.DMA((2,)),
            pltpu.SemaphoreType.DMA((2,)),
        ],
    )(_sc_body)
    return run(x)


def kernel(perturb, user_emb, item_emb):
    del perturb
    it = item_emb.T
    i_bulk = _sc_copy(it)
    u = _tc_copy(user_emb.T, 57344).T
    i = _tail_patch(i_bulk, it, it.shape[1] // _LANE).T
    return (u, i)
